# baseline (speedup 1.0000x reference)
import functools

import jax
import jax.numpy as jnp
from jax import lax
from jax.experimental import pallas as pl
from jax.experimental.pallas import tpu as pltpu
from jax.experimental.pallas import tpu_sc as plsc

E = 8
K = 2
H = 1024
I = 2048
T = 2048
BLK = 128
NB = (T * K) // BLK + (E - 1)
TE_PAD = NB * BLK


def _mlp_body(be_ref, x_ref, w1_ref, w3_ref, w2_ref, o_ref):
    x = x_ref[...]
    h1 = jax.lax.dot_general(x, w1_ref[...], (((1,), (1,)), ((), ())),
                             preferred_element_type=jnp.float32)
    h3 = jax.lax.dot_general(x, w3_ref[...], (((1,), (1,)), ((), ())),
                             preferred_element_type=jnp.float32)
    h = h1 * jax.nn.sigmoid(h1) * h3
    o_ref[...] = jax.lax.dot_general(h, w2_ref[...], (((1,), (1,)), ((), ())),
                                     preferred_element_type=jnp.float32)


def _grouped_mlp(block_expert, x_sorted, w1, w3, w2):
    grid_spec = pltpu.PrefetchScalarGridSpec(
        num_scalar_prefetch=1,
        grid=(NB,),
        in_specs=[
            pl.BlockSpec((BLK, H), lambda i, be: (i, 0)),
            pl.BlockSpec((None, I, H), lambda i, be: (be[i], 0, 0)),
            pl.BlockSpec((None, I, H), lambda i, be: (be[i], 0, 0)),
            pl.BlockSpec((None, H, I), lambda i, be: (be[i], 0, 0)),
        ],
        out_specs=pl.BlockSpec((BLK, H), lambda i, be: (i, 0)),
    )
    return pl.pallas_call(
        _mlp_body,
        grid_spec=grid_spec,
        out_shape=jax.ShapeDtypeStruct((TE_PAD, H), jnp.float32),
    )(block_expert, x_sorted, w1, w3, w2)


def _sel_by(arr, idx, iota_e):
    return jnp.sum(jnp.where(iota_e == idx, arr, 0.0), axis=0, keepdims=True)


def _route_body(x_ref, g_ref, pos_ref, w_ref, be_ref):
    lT = jax.lax.dot_general(g_ref[...], x_ref[...], (((1,), (1,)), ((), ())),
                             preferred_element_type=jnp.float32)
    m = jnp.max(lT, axis=0, keepdims=True)
    p = jnp.exp(lT - m)
    p = p / jnp.sum(p, axis=0, keepdims=True)
    iota_e = jax.lax.broadcasted_iota(jnp.int32, (E, T), 0)
    m1 = jnp.max(p, axis=0, keepdims=True)
    a1 = jnp.min(jnp.where(p == m1, iota_e, E), axis=0, keepdims=True)
    p2 = jnp.where(iota_e == a1, -1.0, p)
    m2 = jnp.max(p2, axis=0, keepdims=True)
    a2 = jnp.min(jnp.where(p2 == m2, iota_e, E), axis=0, keepdims=True)
    s = m1 + m2
    w_ref[0:1, :] = m1 / s
    w_ref[1:2, :] = m2 / s
    oh0 = (iota_e == a1).astype(jnp.float32)
    oh1 = (iota_e == a2).astype(jnp.float32)
    c01 = oh0 + oh1
    cs = c01
    d = 1
    while d < T:
        cs = cs + jnp.concatenate(
            [jnp.zeros((E, d), jnp.float32), cs[:, :T - d]], axis=1)
        d *= 2
    excl = cs - c01
    counts = cs[:, T - 1:T]
    padded = jnp.floor((counts + (BLK - 1)) / BLK) * BLK
    cp = padded
    d = 1
    while d < E:
        cp = cp + jnp.concatenate(
            [jnp.zeros((d, 1), jnp.float32), cp[:E - d]], axis=0)
        d *= 2
    gstart = jnp.broadcast_to(cp - padded, (E, T))
    pos0 = _sel_by(gstart, a1, iota_e) + _sel_by(excl, a1, iota_e)
    pos1 = _sel_by(gstart, a2, iota_e) + _sel_by(excl, a2, iota_e) + \
        _sel_by(oh0, a2, iota_e)
    pos_ref[0:1, :] = pos0.astype(jnp.int32)
    pos_ref[1:2, :] = pos1.astype(jnp.int32)
    bstart = jax.lax.broadcasted_iota(jnp.int32, (1, NB), 1).astype(jnp.float32) * BLK
    be = jnp.sum((cp <= bstart).astype(jnp.float32), axis=0, keepdims=True)
    iota_e1 = jax.lax.broadcasted_iota(jnp.int32, (E, 1), 0).astype(jnp.float32)
    last_e = jnp.max(jnp.where(counts > 0, iota_e1, -1.0), axis=0, keepdims=True)
    be = jnp.where(bstart < cp[E - 1:E, :], be, last_e)
    be_ref[...] = be.astype(jnp.int32)


def _route(x, gate_w):
    return pl.pallas_call(
        _route_body,
        out_shape=(
            jax.ShapeDtypeStruct((K, T), jnp.int32),
            jax.ShapeDtypeStruct((K, T), jnp.float32),
            jax.ShapeDtypeStruct((1, NB), jnp.int32),
        ),
    )(x, gate_w)


_DEBUG_JAX_COMBINE = False
_SC_MESH = plsc.VectorSubcoreMesh(core_axis_name="c", subcore_axis_name="s")
NW = 32
PW = (T * K) // NW
CH = 16


def _dispatch_body(x_hbm, pos_hbm, xs_hbm, posv, xbuf, sem):
    wid = lax.axis_index("s") * 2 + lax.axis_index("c")
    base = wid * PW
    tok_base = (wid % 16) * PW
    pltpu.sync_copy(pos_hbm.at[pl.ds(base, PW)], posv)
    for c in range(PW // CH):
        pltpu.sync_copy(x_hbm.at[pl.ds(tok_base + c * CH, CH)], xbuf)
        idx = posv[pl.ds(c * CH, 16)]
        pltpu.async_copy(xbuf, xs_hbm.at[idx], sem).wait()


def _sc_dispatch(x, pos_flat):
    return pl.kernel(
        _dispatch_body,
        out_type=jax.ShapeDtypeStruct((TE_PAD, H), jnp.float32),
        mesh=_SC_MESH,
        compiler_params=pltpu.CompilerParams(needs_layout_passes=False),
        scratch_types=[
            pltpu.VMEM((PW,), jnp.int32),
            pltpu.VMEM((CH, H), jnp.float32),
            pltpu.SemaphoreType.DMA,
        ],
    )(x, pos_flat)


TPW = T // NW


def _combine_body(os_hbm, pos_hbm, w_hbm, out_hbm, p0v, p1v, w0v, w1v,
                  r0, r1, ob, sem0, sem1):
    wid = lax.axis_index("s") * 2 + lax.axis_index("c")
    tok_base = wid * TPW
    pltpu.sync_copy(pos_hbm.at[pl.ds(tok_base, TPW)], p0v)
    pltpu.sync_copy(pos_hbm.at[pl.ds(T + tok_base, TPW)], p1v)
    pltpu.sync_copy(w_hbm.at[pl.ds(tok_base, TPW)], w0v)
    pltpu.sync_copy(w_hbm.at[pl.ds(T + tok_base, TPW)], w1v)
    for c in range(TPW // CH):
        i0 = p0v[pl.ds(c * CH, 16)]
        i1 = p1v[pl.ds(c * CH, 16)]
        cp0 = pltpu.async_copy(os_hbm.at[i0], r0, sem0)
        cp1 = pltpu.async_copy(os_hbm.at[i1], r1, sem1)
        cp0.wait()
        cp1.wait()
        wc0 = w0v[pl.ds(c * CH, 16)]
        wc1 = w1v[pl.ds(c * CH, 16)]
        for r in range(CH):
            s0 = jnp.broadcast_to(wc0[r], (16,))
            s1 = jnp.broadcast_to(wc1[r], (16,))

            def add(j, _):
                ob[r, pl.ds(j * 16, 16)] = (
                    s0 * r0[r, pl.ds(j * 16, 16)] +
                    s1 * r1[r, pl.ds(j * 16, 16)])
                return 0
            lax.fori_loop(0, H // 16, add, 0)
        pltpu.sync_copy(ob, out_hbm.at[pl.ds(tok_base + c * CH, CH)])


def _sc_combine(out_sorted, pos_flat, w_flat):
    return pl.kernel(
        _combine_body,
        out_type=jax.ShapeDtypeStruct((T, H), jnp.float32),
        mesh=_SC_MESH,
        compiler_params=pltpu.CompilerParams(needs_layout_passes=False),
        scratch_types=[
            pltpu.VMEM((TPW,), jnp.int32),
            pltpu.VMEM((TPW,), jnp.int32),
            pltpu.VMEM((TPW,), jnp.float32),
            pltpu.VMEM((TPW,), jnp.float32),
            pltpu.VMEM((CH, H), jnp.float32),
            pltpu.VMEM((CH, H), jnp.float32),
            pltpu.VMEM((CH, H), jnp.float32),
            pltpu.SemaphoreType.DMA,
            pltpu.SemaphoreType.DMA,
        ],
    )(out_sorted, pos_flat, w_flat)


def kernel(hidden_states, gate_w, w1_weight, w3_weight, w2_weight):
    x = hidden_states
    pos_T, w_T, be = _route(x, gate_w)
    pos_flat = pos_T.reshape(T * K)
    w_flat = w_T.reshape(T * K)
    block_expert = be.reshape(NB)

    x_sorted = _sc_dispatch(x, pos_flat)
    out_sorted = _grouped_mlp(block_expert, x_sorted,
                              w1_weight, w3_weight, w2_weight)
    if _DEBUG_JAX_COMBINE:
        o0 = jnp.take(out_sorted, pos_flat[:T], axis=0) * w_flat[:T, None]
        o1 = jnp.take(out_sorted, pos_flat[T:], axis=0) * w_flat[T:, None]
        jax_w = o0 + o1
        sw = _sc_combine(out_sorted, pos_flat, w_flat)
        return (jax_w + 10.0 * (sw - jax_w)).astype(hidden_states.dtype)
    return _sc_combine(out_sorted, pos_flat, w_flat).astype(hidden_states.dtype)


# speedup vs baseline: 1.4383x; 1.4383x over previous
import functools

import jax
import jax.numpy as jnp
from jax import lax
from jax.experimental import pallas as pl
from jax.experimental.pallas import tpu as pltpu
from jax.experimental.pallas import tpu_sc as plsc

E = 8
K = 2
H = 1024
I = 2048
T = 2048
BLK = 512
NB = (T * K) // BLK + (E - 1)
TE_PAD = NB * BLK


def _mlp_body(be_ref, x_ref, w1_ref, w3_ref, w2_ref, o_ref):
    x = x_ref[...]
    h1 = jax.lax.dot_general(x, w1_ref[...], (((1,), (1,)), ((), ())),
                             preferred_element_type=jnp.float32)
    h3 = jax.lax.dot_general(x, w3_ref[...], (((1,), (1,)), ((), ())),
                             preferred_element_type=jnp.float32)
    h = h1 * jax.nn.sigmoid(h1) * h3
    o_ref[...] = jax.lax.dot_general(h, w2_ref[...], (((1,), (1,)), ((), ())),
                                     preferred_element_type=jnp.float32)


def _grouped_mlp(block_expert, x_sorted, w1, w3, w2):
    grid_spec = pltpu.PrefetchScalarGridSpec(
        num_scalar_prefetch=1,
        grid=(NB,),
        in_specs=[
            pl.BlockSpec((BLK, H), lambda i, be: (i, 0)),
            pl.BlockSpec((None, I, H), lambda i, be: (be[i], 0, 0)),
            pl.BlockSpec((None, I, H), lambda i, be: (be[i], 0, 0)),
            pl.BlockSpec((None, H, I), lambda i, be: (be[i], 0, 0)),
        ],
        out_specs=pl.BlockSpec((BLK, H), lambda i, be: (i, 0)),
    )
    return pl.pallas_call(
        _mlp_body,
        grid_spec=grid_spec,
        out_shape=jax.ShapeDtypeStruct((TE_PAD, H), jnp.float32),
        compiler_params=pltpu.CompilerParams(
            vmem_limit_bytes=100 * 1024 * 1024),
    )(block_expert, x_sorted, w1, w3, w2)


def _sel_by(arr, idx, iota_e):
    return jnp.sum(jnp.where(iota_e == idx, arr, 0.0), axis=0, keepdims=True)


def _route_body(x_ref, g_ref, pos_ref, w_ref, be_ref):
    lT = jax.lax.dot_general(g_ref[...], x_ref[...], (((1,), (1,)), ((), ())),
                             preferred_element_type=jnp.float32)
    m = jnp.max(lT, axis=0, keepdims=True)
    p = jnp.exp(lT - m)
    p = p / jnp.sum(p, axis=0, keepdims=True)
    iota_e = jax.lax.broadcasted_iota(jnp.int32, (E, T), 0)
    m1 = jnp.max(p, axis=0, keepdims=True)
    a1 = jnp.min(jnp.where(p == m1, iota_e, E), axis=0, keepdims=True)
    p2 = jnp.where(iota_e == a1, -1.0, p)
    m2 = jnp.max(p2, axis=0, keepdims=True)
    a2 = jnp.min(jnp.where(p2 == m2, iota_e, E), axis=0, keepdims=True)
    s = m1 + m2
    w_ref[0:1, :] = m1 / s
    w_ref[1:2, :] = m2 / s
    oh0 = (iota_e == a1).astype(jnp.float32)
    oh1 = (iota_e == a2).astype(jnp.float32)
    c01 = oh0 + oh1
    cs = c01
    d = 1
    while d < T:
        cs = cs + jnp.concatenate(
            [jnp.zeros((E, d), jnp.float32), cs[:, :T - d]], axis=1)
        d *= 2
    excl = cs - c01
    counts = cs[:, T - 1:T]
    padded = jnp.floor((counts + (BLK - 1)) / BLK) * BLK
    cp = padded
    d = 1
    while d < E:
        cp = cp + jnp.concatenate(
            [jnp.zeros((d, 1), jnp.float32), cp[:E - d]], axis=0)
        d *= 2
    gstart = jnp.broadcast_to(cp - padded, (E, T))
    pos0 = _sel_by(gstart, a1, iota_e) + _sel_by(excl, a1, iota_e)
    pos1 = _sel_by(gstart, a2, iota_e) + _sel_by(excl, a2, iota_e) + \
        _sel_by(oh0, a2, iota_e)
    pos_ref[0:1, :] = pos0.astype(jnp.int32)
    pos_ref[1:2, :] = pos1.astype(jnp.int32)
    bstart = jax.lax.broadcasted_iota(jnp.int32, (1, NB), 1).astype(jnp.float32) * BLK
    be = jnp.sum((cp <= bstart).astype(jnp.float32), axis=0, keepdims=True)
    iota_e1 = jax.lax.broadcasted_iota(jnp.int32, (E, 1), 0).astype(jnp.float32)
    last_e = jnp.max(jnp.where(counts > 0, iota_e1, -1.0), axis=0, keepdims=True)
    be = jnp.where(bstart < cp[E - 1:E, :], be, last_e)
    be_ref[...] = be.astype(jnp.int32)


def _route(x, gate_w):
    return pl.pallas_call(
        _route_body,
        out_shape=(
            jax.ShapeDtypeStruct((K, T), jnp.int32),
            jax.ShapeDtypeStruct((K, T), jnp.float32),
            jax.ShapeDtypeStruct((1, NB), jnp.int32),
        ),
    )(x, gate_w)


_DEBUG_JAX_COMBINE = False
_SC_MESH = plsc.VectorSubcoreMesh(core_axis_name="c", subcore_axis_name="s")
NW = 32
PW = (T * K) // NW
CH = 16


def _dispatch_body(x_hbm, pos_hbm, xs_hbm, posv, xbuf, sem):
    wid = lax.axis_index("s") * 2 + lax.axis_index("c")
    base = wid * PW
    tok_base = (wid % 16) * PW
    pltpu.sync_copy(pos_hbm.at[pl.ds(base, PW)], posv)
    for c in range(PW // CH):
        pltpu.sync_copy(x_hbm.at[pl.ds(tok_base + c * CH, CH)], xbuf)
        idx = posv[pl.ds(c * CH, 16)]
        pltpu.async_copy(xbuf, xs_hbm.at[idx], sem).wait()


def _sc_dispatch(x, pos_flat):
    return pl.kernel(
        _dispatch_body,
        out_type=jax.ShapeDtypeStruct((TE_PAD, H), jnp.float32),
        mesh=_SC_MESH,
        compiler_params=pltpu.CompilerParams(needs_layout_passes=False),
        scratch_types=[
            pltpu.VMEM((PW,), jnp.int32),
            pltpu.VMEM((CH, H), jnp.float32),
            pltpu.SemaphoreType.DMA,
        ],
    )(x, pos_flat)


TPW = T // NW


def _combine_body(os_hbm, pos_hbm, w_hbm, out_hbm, p0v, p1v, w0v, w1v,
                  r0, r1, ob, sem0, sem1):
    wid = lax.axis_index("s") * 2 + lax.axis_index("c")
    tok_base = wid * TPW
    pltpu.sync_copy(pos_hbm.at[pl.ds(tok_base, TPW)], p0v)
    pltpu.sync_copy(pos_hbm.at[pl.ds(T + tok_base, TPW)], p1v)
    pltpu.sync_copy(w_hbm.at[pl.ds(tok_base, TPW)], w0v)
    pltpu.sync_copy(w_hbm.at[pl.ds(T + tok_base, TPW)], w1v)
    for c in range(TPW // CH):
        i0 = p0v[pl.ds(c * CH, 16)]
        i1 = p1v[pl.ds(c * CH, 16)]
        cp0 = pltpu.async_copy(os_hbm.at[i0], r0, sem0)
        cp1 = pltpu.async_copy(os_hbm.at[i1], r1, sem1)
        cp0.wait()
        cp1.wait()
        wc0 = w0v[pl.ds(c * CH, 16)]
        wc1 = w1v[pl.ds(c * CH, 16)]
        for r in range(CH):
            s0 = jnp.broadcast_to(wc0[r], (16,))
            s1 = jnp.broadcast_to(wc1[r], (16,))

            def add(j, _):
                ob[r, pl.ds(j * 16, 16)] = (
                    s0 * r0[r, pl.ds(j * 16, 16)] +
                    s1 * r1[r, pl.ds(j * 16, 16)])
                return 0
            lax.fori_loop(0, H // 16, add, 0)
        pltpu.sync_copy(ob, out_hbm.at[pl.ds(tok_base + c * CH, CH)])


def _sc_combine(out_sorted, pos_flat, w_flat):
    return pl.kernel(
        _combine_body,
        out_type=jax.ShapeDtypeStruct((T, H), jnp.float32),
        mesh=_SC_MESH,
        compiler_params=pltpu.CompilerParams(needs_layout_passes=False),
        scratch_types=[
            pltpu.VMEM((TPW,), jnp.int32),
            pltpu.VMEM((TPW,), jnp.int32),
            pltpu.VMEM((TPW,), jnp.float32),
            pltpu.VMEM((TPW,), jnp.float32),
            pltpu.VMEM((CH, H), jnp.float32),
            pltpu.VMEM((CH, H), jnp.float32),
            pltpu.VMEM((CH, H), jnp.float32),
            pltpu.SemaphoreType.DMA,
            pltpu.SemaphoreType.DMA,
        ],
    )(out_sorted, pos_flat, w_flat)


def kernel(hidden_states, gate_w, w1_weight, w3_weight, w2_weight):
    x = hidden_states
    pos_T, w_T, be = _route(x, gate_w)
    pos_flat = pos_T.reshape(T * K)
    w_flat = w_T.reshape(T * K)
    block_expert = be.reshape(NB)

    x_sorted = _sc_dispatch(x, pos_flat)
    out_sorted = _grouped_mlp(block_expert, x_sorted,
                              w1_weight, w3_weight, w2_weight)
    if _DEBUG_JAX_COMBINE:
        o0 = jnp.take(out_sorted, pos_flat[:T], axis=0) * w_flat[:T, None]
        o1 = jnp.take(out_sorted, pos_flat[T:], axis=0) * w_flat[T:, None]
        jax_w = o0 + o1
        sw = _sc_combine(out_sorted, pos_flat, w_flat)
        return (jax_w + 10.0 * (sw - jax_w)).astype(hidden_states.dtype)
    return _sc_combine(out_sorted, pos_flat, w_flat).astype(hidden_states.dtype)


# speedup vs baseline: 1.5149x; 1.0533x over previous
import functools

import jax
import jax.numpy as jnp
from jax import lax
from jax.experimental import pallas as pl
from jax.experimental.pallas import tpu as pltpu
from jax.experimental.pallas import tpu_sc as plsc

E = 8
K = 2
H = 1024
I = 2048
T = 2048
BLK = 512
NB = (T * K) // BLK + (E - 1)
TE_PAD = NB * BLK


def _mlp_body(be_ref, x_ref, w1_ref, w3_ref, w2_ref, o_ref):
    x = x_ref[...]
    h1 = jax.lax.dot_general(x, w1_ref[...], (((1,), (1,)), ((), ())),
                             preferred_element_type=jnp.float32)
    h3 = jax.lax.dot_general(x, w3_ref[...], (((1,), (1,)), ((), ())),
                             preferred_element_type=jnp.float32)
    h = h1 * jax.nn.sigmoid(h1) * h3
    o_ref[...] = jax.lax.dot_general(h, w2_ref[...], (((1,), (1,)), ((), ())),
                                     preferred_element_type=jnp.float32)


def _grouped_mlp(block_expert, x_sorted, w1, w3, w2):
    grid_spec = pltpu.PrefetchScalarGridSpec(
        num_scalar_prefetch=1,
        grid=(NB,),
        in_specs=[
            pl.BlockSpec((BLK, H), lambda i, be: (i, 0)),
            pl.BlockSpec((None, I, H), lambda i, be: (be[i], 0, 0)),
            pl.BlockSpec((None, I, H), lambda i, be: (be[i], 0, 0)),
            pl.BlockSpec((None, H, I), lambda i, be: (be[i], 0, 0)),
        ],
        out_specs=pl.BlockSpec((BLK, H), lambda i, be: (i, 0)),
    )
    return pl.pallas_call(
        _mlp_body,
        grid_spec=grid_spec,
        out_shape=jax.ShapeDtypeStruct((TE_PAD, H), jnp.float32),
        compiler_params=pltpu.CompilerParams(
            vmem_limit_bytes=100 * 1024 * 1024),
    )(block_expert, x_sorted, w1, w3, w2)


def _sel_by(arr, idx, iota_e):
    return jnp.sum(jnp.where(iota_e == idx, arr, 0.0), axis=0, keepdims=True)


def _route_body(x_ref, g_ref, pos_ref, w_ref, be_ref):
    lT = jax.lax.dot_general(g_ref[...], x_ref[...], (((1,), (1,)), ((), ())),
                             preferred_element_type=jnp.float32)
    m = jnp.max(lT, axis=0, keepdims=True)
    p = jnp.exp(lT - m)
    p = p / jnp.sum(p, axis=0, keepdims=True)
    iota_e = jax.lax.broadcasted_iota(jnp.int32, (E, T), 0)
    m1 = jnp.max(p, axis=0, keepdims=True)
    a1 = jnp.min(jnp.where(p == m1, iota_e, E), axis=0, keepdims=True)
    p2 = jnp.where(iota_e == a1, -1.0, p)
    m2 = jnp.max(p2, axis=0, keepdims=True)
    a2 = jnp.min(jnp.where(p2 == m2, iota_e, E), axis=0, keepdims=True)
    s = m1 + m2
    w_ref[0:1, :] = m1 / s
    w_ref[1:2, :] = m2 / s
    oh0 = (iota_e == a1).astype(jnp.float32)
    oh1 = (iota_e == a2).astype(jnp.float32)
    c01 = oh0 + oh1
    cs = c01
    d = 1
    while d < T:
        cs = cs + jnp.concatenate(
            [jnp.zeros((E, d), jnp.float32), cs[:, :T - d]], axis=1)
        d *= 2
    excl = cs - c01
    counts = cs[:, T - 1:T]
    padded = jnp.floor((counts + (BLK - 1)) / BLK) * BLK
    cp = padded
    d = 1
    while d < E:
        cp = cp + jnp.concatenate(
            [jnp.zeros((d, 1), jnp.float32), cp[:E - d]], axis=0)
        d *= 2
    gstart = jnp.broadcast_to(cp - padded, (E, T))
    pos0 = _sel_by(gstart, a1, iota_e) + _sel_by(excl, a1, iota_e)
    pos1 = _sel_by(gstart, a2, iota_e) + _sel_by(excl, a2, iota_e) + \
        _sel_by(oh0, a2, iota_e)
    pos_ref[0:1, :] = pos0.astype(jnp.int32)
    pos_ref[1:2, :] = pos1.astype(jnp.int32)
    bstart = jax.lax.broadcasted_iota(jnp.int32, (1, NB), 1).astype(jnp.float32) * BLK
    be = jnp.sum((cp <= bstart).astype(jnp.float32), axis=0, keepdims=True)
    iota_e1 = jax.lax.broadcasted_iota(jnp.int32, (E, 1), 0).astype(jnp.float32)
    last_e = jnp.max(jnp.where(counts > 0, iota_e1, -1.0), axis=0, keepdims=True)
    be = jnp.where(bstart < cp[E - 1:E, :], be, last_e)
    be_ref[...] = be.astype(jnp.int32)


def _route(x, gate_w):
    return pl.pallas_call(
        _route_body,
        out_shape=(
            jax.ShapeDtypeStruct((K, T), jnp.int32),
            jax.ShapeDtypeStruct((K, T), jnp.float32),
            jax.ShapeDtypeStruct((1, NB), jnp.int32),
        ),
    )(x, gate_w)


_DEBUG_JAX_COMBINE = False
_SC_MESH = plsc.VectorSubcoreMesh(core_axis_name="c", subcore_axis_name="s")
NW = 32
PW = (T * K) // NW
CH = 16


def _dispatch_body(x_hbm, pos_hbm, xs_hbm, posv, xb0, xb1,
                   gs0, gs1, ss0, ss1):
    wid = lax.axis_index("s") * 2 + lax.axis_index("c")
    base = wid * PW
    tok_base = (wid % 16) * PW
    pltpu.sync_copy(pos_hbm.at[pl.ds(base, PW)], posv)
    xb, gs, ss = (xb0, xb1), (gs0, gs1), (ss0, ss1)
    nch = PW // CH
    grabs, scats = {}, {}

    def start_gather(c):
        grabs[c] = pltpu.async_copy(
            x_hbm.at[pl.ds(tok_base + c * CH, CH)], xb[c % 2], gs[c % 2])

    start_gather(0)
    for c in range(nch):
        if c + 1 < nch:
            if c >= 1:
                scats[c - 1].wait()
            start_gather(c + 1)
        grabs[c].wait()
        idx = posv[pl.ds(c * CH, 16)]
        scats[c] = pltpu.async_copy(xb[c % 2], xs_hbm.at[idx], ss[c % 2])
    scats[nch - 2].wait()
    scats[nch - 1].wait()


def _sc_dispatch(x, pos_flat):
    return pl.kernel(
        _dispatch_body,
        out_type=jax.ShapeDtypeStruct((TE_PAD, H), jnp.float32),
        mesh=_SC_MESH,
        compiler_params=pltpu.CompilerParams(needs_layout_passes=False),
        scratch_types=[
            pltpu.VMEM((PW,), jnp.int32),
            pltpu.VMEM((CH, H), jnp.float32),
            pltpu.VMEM((CH, H), jnp.float32),
            pltpu.SemaphoreType.DMA,
            pltpu.SemaphoreType.DMA,
            pltpu.SemaphoreType.DMA,
            pltpu.SemaphoreType.DMA,
        ],
    )(x, pos_flat)


TPW = T // NW


def _combine_body(os_hbm, pos_hbm, w_hbm, out_hbm, p0v, p1v, w0v, w1v,
                  r0, r1, r0x, r1x, ob, sem0, sem1):
    wid = lax.axis_index("s") * 2 + lax.axis_index("c")
    tok_base = wid * TPW
    pltpu.sync_copy(pos_hbm.at[pl.ds(tok_base, TPW)], p0v)
    pltpu.sync_copy(pos_hbm.at[pl.ds(T + tok_base, TPW)], p1v)
    pltpu.sync_copy(w_hbm.at[pl.ds(tok_base, TPW)], w0v)
    pltpu.sync_copy(w_hbm.at[pl.ds(T + tok_base, TPW)], w1v)
    r0b, r1b, sems = (r0, r0x), (r1, r1x), (sem0, sem1)
    nch = TPW // CH
    cps = {}

    def start_gathers(c):
        i0 = p0v[pl.ds(c * CH, 16)]
        i1 = p1v[pl.ds(c * CH, 16)]
        cps[c] = (pltpu.async_copy(os_hbm.at[i0], r0b[c % 2], sems[c % 2]),
                  pltpu.async_copy(os_hbm.at[i1], r1b[c % 2], sems[c % 2]))

    start_gathers(0)
    for c in range(nch):
        if c + 1 < nch:
            start_gathers(c + 1)
        cps[c][0].wait()
        cps[c][1].wait()
        wc0 = w0v[pl.ds(c * CH, 16)]
        wc1 = w1v[pl.ds(c * CH, 16)]
        for r in range(CH):
            s0 = jnp.broadcast_to(wc0[r], (16,))
            s1 = jnp.broadcast_to(wc1[r], (16,))

            def add(j, _):
                ob[r, pl.ds(j * 16, 16)] = (
                    s0 * r0b[c % 2][r, pl.ds(j * 16, 16)] +
                    s1 * r1b[c % 2][r, pl.ds(j * 16, 16)])
                return 0
            lax.fori_loop(0, H // 16, add, 0)
        pltpu.sync_copy(ob, out_hbm.at[pl.ds(tok_base + c * CH, CH)])


def _sc_combine(out_sorted, pos_flat, w_flat):
    return pl.kernel(
        _combine_body,
        out_type=jax.ShapeDtypeStruct((T, H), jnp.float32),
        mesh=_SC_MESH,
        compiler_params=pltpu.CompilerParams(needs_layout_passes=False),
        scratch_types=[
            pltpu.VMEM((TPW,), jnp.int32),
            pltpu.VMEM((TPW,), jnp.int32),
            pltpu.VMEM((TPW,), jnp.float32),
            pltpu.VMEM((TPW,), jnp.float32),
            pltpu.VMEM((CH, H), jnp.float32),
            pltpu.VMEM((CH, H), jnp.float32),
            pltpu.VMEM((CH, H), jnp.float32),
            pltpu.VMEM((CH, H), jnp.float32),
            pltpu.VMEM((CH, H), jnp.float32),
            pltpu.SemaphoreType.DMA,
            pltpu.SemaphoreType.DMA,
        ],
    )(out_sorted, pos_flat, w_flat)


def kernel(hidden_states, gate_w, w1_weight, w3_weight, w2_weight):
    x = hidden_states
    pos_T, w_T, be = _route(x, gate_w)
    pos_flat = pos_T.reshape(T * K)
    w_flat = w_T.reshape(T * K)
    block_expert = be.reshape(NB)

    x_sorted = _sc_dispatch(x, pos_flat)
    out_sorted = _grouped_mlp(block_expert, x_sorted,
                              w1_weight, w3_weight, w2_weight)
    if _DEBUG_JAX_COMBINE:
        o0 = jnp.take(out_sorted, pos_flat[:T], axis=0) * w_flat[:T, None]
        o1 = jnp.take(out_sorted, pos_flat[T:], axis=0) * w_flat[T:, None]
        jax_w = o0 + o1
        sw = _sc_combine(out_sorted, pos_flat, w_flat)
        return (jax_w + 10.0 * (sw - jax_w)).astype(hidden_states.dtype)
    return _sc_combine(out_sorted, pos_flat, w_flat).astype(hidden_states.dtype)
